# baseline (speedup 1.0000x reference)
import functools
import math

import jax
import jax.numpy as jnp
from jax import lax
from jax.experimental import pallas as pl
from jax.experimental.pallas import tpu as pltpu
from jax.experimental.pallas import tpu_sc as plsc

VOCAB = 8007
SEQ = 512
DIM = 768
DEPTH = 6
HEADS = 8
DHEAD = DIM // HEADS
FF = 3072
CHUNK = 64
NCH = SEQ // CHUNK
B = 8
FCH = 12
FCW = FF // FCH

_NC = 2
_NS = 16
_NW = _NC * _NS
_TOKENS = B * SEQ
_ROWS_PER_W = _TOKENS // _NW


def _emb_gather_body(table_hbm, idx_hbm, out_hbm, idx_v, rows_v, sem):
    wid = lax.axis_index("s") * _NC + lax.axis_index("c")
    base = wid * _ROWS_PER_W
    pltpu.sync_copy(idx_hbm.at[pl.ds(base, _ROWS_PER_W)], idx_v)
    pltpu.async_copy(table_hbm.at[idx_v], rows_v, sem).wait()
    pltpu.sync_copy(rows_v, out_hbm.at[pl.ds(base, _ROWS_PER_W)])


_emb_gather = functools.partial(
    pl.kernel,
    out_type=jax.ShapeDtypeStruct((_TOKENS, DIM), jnp.float32),
    mesh=plsc.VectorSubcoreMesh(core_axis_name="c", subcore_axis_name="s"),
    scratch_types=[
        pltpu.VMEM((_ROWS_PER_W,), jnp.int32),
        pltpu.VMEM((_ROWS_PER_W, DIM), jnp.float32),
        pltpu.SemaphoreType.DMA,
    ],
)(_emb_gather_body)


def _ln(x, g, b):
    m = jnp.mean(x, axis=-1, keepdims=True)
    d = x - m
    v = jnp.mean(d * d, axis=-1, keepdims=True)
    return d * lax.rsqrt(v + 1e-12) * g + b


def _dg3(a, b, dims):
    return lax.dot_general(a.astype(jnp.bfloat16), b.astype(jnp.bfloat16),
                           dims, preferred_element_type=jnp.float32)


_MM = (((1,), (0,)), ((), ()))


def _mm3(a, b):
    return _dg3(a, b, _MM)


def _embed_body(rows, pos, seg, sidf, out):
    sid = sidf[0].astype(jnp.float32)
    seg0 = seg[0:1, :]
    seg1 = seg[1:2, :]
    out[0] = rows[0] + pos[...] + seg0 + sid * (seg1 - seg0)


def _embed_add(rows, pos, seg, sidf):
    return pl.pallas_call(
        _embed_body,
        grid=(B,),
        in_specs=[
            pl.BlockSpec((1, SEQ, DIM), lambda b: (b, 0, 0)),
            pl.BlockSpec((SEQ, DIM), lambda b: (0, 0)),
            pl.BlockSpec((2, DIM), lambda b: (0, 0)),
            pl.BlockSpec((1, SEQ, 1), lambda b: (b, 0, 0)),
        ],
        out_specs=pl.BlockSpec((1, SEQ, DIM), lambda b: (b, 0, 0)),
        out_shape=jax.ShapeDtypeStruct((B, SEQ, DIM), jnp.float32),
    )(rows, pos, seg, sidf)


def _transformer_body(x0, wq, wk, wv, wo, w1, b1, w2, b2,
                      g1, be1, g2, be2, gf, bf, qaw, qab, out, x_scr, h2_scr):
    l = pl.program_id(0)
    c = pl.program_id(1)
    b = pl.program_id(2)

    @pl.when((l == 0) & (c == 0))
    def _init():
        x_scr[b] = x0[0]

    @pl.when(c == 0)
    def _attention():
        x = x_scr[b]

        h = _ln(x, g1[0], be1[0])
        scale = jnp.float32(1.0 / math.sqrt(DHEAD))
        q = _mm3(h, wq[0]) * scale
        k = _mm3(h, wk[0])
        v = _mm3(h, wv[0])

        def win(a):
            a3 = a.reshape(NCH, CHUNK, DIM)
            z = jnp.zeros((1, CHUNK, DIM), a.dtype)
            return jnp.concatenate([jnp.concatenate([z, a3[:-1]], 0), a3], 1)

        q3h = q.astype(jnp.bfloat16).reshape(NCH, CHUNK, DIM)
        kkh = win(k.astype(jnp.bfloat16))
        vvh = win(v.astype(jnp.bfloat16))

        nn = lax.broadcasted_iota(jnp.int32, (NCH, CHUNK, 2 * CHUNK), 0)
        qi = lax.broadcasted_iota(jnp.int32, (NCH, CHUNK, 2 * CHUNK), 1)
        ki = lax.broadcasted_iota(jnp.int32, (NCH, CHUNK, 2 * CHUNK), 2)
        valid = (ki - CHUNK <= qi) & ((nn > 0) | (ki >= CHUNK))
        neg = jnp.float32(-1e9)

        dq = (((2,), (2,)), ((0,), (0,)))
        dv = (((2,), (1,)), ((0,), (0,)))
        fq = lambda u, v: lax.dot_general(u, v, dq,
                                          preferred_element_type=jnp.float32)
        fv = lambda u, v: lax.dot_general(u, v, dv,
                                          preferred_element_type=jnp.float32)
        outs = []
        for hh in range(HEADS):
            sl = slice(hh * DHEAD, (hh + 1) * DHEAD)
            s = fq(q3h[:, :, sl], kkh[:, :, sl])
            s = jnp.where(valid, s, neg)
            m = jnp.max(s, axis=-1, keepdims=True)
            e = jnp.exp(s - m)
            a = e / jnp.sum(e, axis=-1, keepdims=True)
            outs.append(fv(a.astype(jnp.bfloat16), vvh[:, :, sl]))
        att = jnp.concatenate(outs, axis=2).reshape(SEQ, DIM)

        x = x + _mm3(att, wo[0])

        h2_scr[b] = _ln(x, g2[0], be2[0])
        x_scr[b] = x + b2[0]

    h2 = h2_scr[b]
    ffa = _mm3(h2, w1[0]) + b1[0]
    ff = jax.nn.gelu(ffa)
    x_scr[b] = x_scr[b] + _mm3(ff, w2[0])

    @pl.when((l == DEPTH - 1) & (c == FCH - 1))
    def _final():
        xf = _ln(x_scr[b], gf[...], bf[...])
        lp = jnp.dot(xf, qaw[...], preferred_element_type=jnp.float32) + qab[...]
        out[0] = jnp.transpose(lp[:, 0:2])


def kernel(input_ids, segments_ids, tok_emb, pos_emb, seg_emb, Wq, Wk, Wv, Wo,
           W1, b1, W2, b2, ln1_g, ln1_b, ln2_g, ln2_b, lnf_g, lnf_b, qa_W, qa_b):
    ids = input_ids.reshape(_TOKENS).astype(jnp.int32)
    rows = _emb_gather(tok_emb, ids).reshape(B, SEQ, DIM)
    sidf = segments_ids.astype(jnp.bfloat16).reshape(B, SEQ, 1)
    x0 = _embed_add(rows, pos_emb, seg_emb, sidf)

    qa_Wp = jnp.zeros((DIM, 8), jnp.float32).at[:, 0:2].set(qa_W)
    qa_bp = jnp.zeros((1, 8), jnp.float32).at[:, 0:2].set(qa_b[None, :])

    def first_only(l, c, b):
        return (jnp.where((l == 0) & (c == 0), b, B - 1), 0, 0)

    const3 = lambda d1, d2: pl.BlockSpec((1, d1, d2), lambda l, c, b: (0, 0, 0))
    perl3 = lambda d1, d2: pl.BlockSpec((1, d1, d2), lambda l, c, b: (l, 0, 0))
    perb3 = lambda d1, d2: pl.BlockSpec((1, d1, d2), lambda l, c, b: (b, 0, 0))
    full2 = lambda d1, d2: pl.BlockSpec((d1, d2), lambda l, c, b: (0, 0))

    logits = pl.pallas_call(
        _transformer_body,
        grid=(DEPTH, FCH, B),
        in_specs=[
            pl.BlockSpec((1, SEQ, DIM), first_only),
            perl3(DIM, DIM),
            perl3(DIM, DIM),
            perl3(DIM, DIM),
            perl3(DIM, DIM),
            pl.BlockSpec((1, DIM, FCW), lambda l, c, b: (l, 0, c)),
            pl.BlockSpec((1, 1, FCW), lambda l, c, b: (l, 0, c)),
            pl.BlockSpec((1, FCW, DIM), lambda l, c, b: (l, c, 0)),
            perl3(1, DIM),
            perl3(1, DIM),
            perl3(1, DIM),
            perl3(1, DIM),
            perl3(1, DIM),
            full2(1, DIM),
            full2(1, DIM),
            full2(DIM, 8),
            full2(1, 8),
        ],
        out_specs=pl.BlockSpec((1, 2, SEQ), lambda l, c, b: (b, 0, 0)),
        out_shape=jax.ShapeDtypeStruct((B, 2, SEQ), jnp.float32),
        scratch_shapes=[pltpu.VMEM((B, SEQ, DIM), jnp.float32),
                        pltpu.VMEM((B, SEQ, DIM), jnp.float32)],
        compiler_params=pltpu.CompilerParams(
            dimension_semantics=("arbitrary", "arbitrary", "arbitrary")),
    )(x0, Wq, Wk, Wv, Wo,
      W1, b1.reshape(DEPTH, 1, FF), W2, b2.reshape(DEPTH, 1, DIM),
      ln1_g.reshape(DEPTH, 1, DIM), ln1_b.reshape(DEPTH, 1, DIM),
      ln2_g.reshape(DEPTH, 1, DIM), ln2_b.reshape(DEPTH, 1, DIM),
      lnf_g.reshape(1, DIM), lnf_b.reshape(1, DIM), qa_Wp, qa_bp)

    return logits[:, 0, :], logits[:, 1, :]


# speedup vs baseline: 1.2281x; 1.2281x over previous
import functools
import math

import jax
import jax.numpy as jnp
from jax import lax
from jax.experimental import pallas as pl
from jax.experimental.pallas import tpu as pltpu
from jax.experimental.pallas import tpu_sc as plsc

VOCAB = 8007
SEQ = 512
DIM = 768
DEPTH = 6
HEADS = 8
DHEAD = DIM // HEADS
FF = 3072
CHUNK = 64
B = 8

_NC = 2
_NS = 16
_NW = _NC * _NS
_TOKENS = B * SEQ
_ROWS_PER_W = _TOKENS // _NW


def _emb_gather_body(table_hbm, idx_hbm, out_hbm, idx_v, rows_v, sem):
    wid = lax.axis_index("s") * _NC + lax.axis_index("c")
    base = wid * _ROWS_PER_W
    pltpu.sync_copy(idx_hbm.at[pl.ds(base, _ROWS_PER_W)], idx_v)
    pltpu.async_copy(table_hbm.at[idx_v], rows_v, sem).wait()
    pltpu.sync_copy(rows_v, out_hbm.at[pl.ds(base, _ROWS_PER_W)])


_emb_gather = functools.partial(
    pl.kernel,
    out_type=jax.ShapeDtypeStruct((_TOKENS, DIM), jnp.float32),
    mesh=plsc.VectorSubcoreMesh(core_axis_name="c", subcore_axis_name="s"),
    scratch_types=[
        pltpu.VMEM((_ROWS_PER_W,), jnp.int32),
        pltpu.VMEM((_ROWS_PER_W, DIM), jnp.float32),
        pltpu.SemaphoreType.DMA,
    ],
)(_emb_gather_body)


def _ln(x, g, b):
    m = jnp.mean(x, axis=-1, keepdims=True)
    d = x - m
    v = jnp.mean(d * d, axis=-1, keepdims=True)
    return d * lax.rsqrt(v + 1e-12) * g + b


def _transformer_body(rows, pos, seg, sidf, wq, wk, wv, wo, w1, b1, w2, b2,
                      g1, be1, g2, be2, gf, bf, qaw, qab, out, x_scr):
    l = pl.program_id(0)
    b = pl.program_id(1)

    @pl.when(l == 0)
    def _init():
        sid = sidf[0]
        seg0 = seg[0:1, :]
        seg1 = seg[1:2, :]
        x_scr[b] = rows[0] + pos[...] + seg0 + sid * (seg1 - seg0)

    x = x_scr[b]

    h = _ln(x, g1[0], be1[0]).astype(jnp.bfloat16)
    q = jnp.dot(h, wq[0], preferred_element_type=jnp.float32)
    k = jnp.dot(h, wk[0], preferred_element_type=jnp.float32)
    v = jnp.dot(h, wv[0], preferred_element_type=jnp.float32)

    ii = lax.broadcasted_iota(jnp.int32, (SEQ, SEQ), 0)
    jj = lax.broadcasted_iota(jnp.int32, (SEQ, SEQ), 1)
    ci = jnp.right_shift(ii, 6)
    cj = jnp.right_shift(jj, 6)
    valid = ((ci == cj) & (jj <= ii)) | (cj + 1 == ci)
    neg = jnp.float32(-1e9)
    scale = jnp.float32(1.0 / math.sqrt(DHEAD))

    outs = []
    for hh in range(HEADS):
        sl = slice(hh * DHEAD, (hh + 1) * DHEAD)
        qh = (q[:, sl] * scale).astype(jnp.bfloat16)
        kh = k[:, sl].astype(jnp.bfloat16)
        vh = v[:, sl].astype(jnp.bfloat16)
        s = lax.dot_general(qh, kh, (((1,), (1,)), ((), ())),
                            preferred_element_type=jnp.float32)
        s = jnp.where(valid, s, neg)
        m = jnp.max(s, axis=-1, keepdims=True)
        e = jnp.exp(s - m)
        a = (e / jnp.sum(e, axis=-1, keepdims=True)).astype(jnp.bfloat16)
        outs.append(jnp.dot(a, vh, preferred_element_type=jnp.float32))
    att = jnp.concatenate(outs, axis=1).astype(jnp.bfloat16)

    x = x + jnp.dot(att, wo[0], preferred_element_type=jnp.float32)

    h2 = _ln(x, g2[0], be2[0]).astype(jnp.bfloat16)
    ffa = jnp.dot(h2, w1[0], preferred_element_type=jnp.float32) + b1[0]
    ff = jax.nn.gelu(ffa).astype(jnp.bfloat16)
    x = x + jnp.dot(ff, w2[0], preferred_element_type=jnp.float32) + b2[0]
    x_scr[b] = x

    @pl.when(l == DEPTH - 1)
    def _final():
        xf = _ln(x, gf[...], bf[...])
        lp = jnp.dot(xf, qaw[...], preferred_element_type=jnp.float32) + qab[...]
        out[0] = lp[:, 0:2]


def kernel(input_ids, segments_ids, tok_emb, pos_emb, seg_emb, Wq, Wk, Wv, Wo,
           W1, b1, W2, b2, ln1_g, ln1_b, ln2_g, ln2_b, lnf_g, lnf_b, qa_W, qa_b):
    ids = input_ids.reshape(_TOKENS).astype(jnp.int32)
    rows = _emb_gather(tok_emb, ids).reshape(B, SEQ, DIM)
    sidf = segments_ids.astype(jnp.float32).reshape(B, SEQ, 1)

    qa_Wp = jnp.zeros((DIM, 128), jnp.float32).at[:, 0:2].set(qa_W)
    qa_bp = jnp.zeros((1, 128), jnp.float32).at[:, 0:2].set(qa_b[None, :])

    const3 = lambda d1, d2: pl.BlockSpec((1, d1, d2), lambda l, b: (0, 0, 0))
    perl3 = lambda d1, d2: pl.BlockSpec((1, d1, d2), lambda l, b: (l, 0, 0))
    perb3 = lambda d1, d2: pl.BlockSpec((1, d1, d2), lambda l, b: (b, 0, 0))
    full2 = lambda d1, d2: pl.BlockSpec((d1, d2), lambda l, b: (0, 0))

    logits = pl.pallas_call(
        _transformer_body,
        grid=(DEPTH, B),
        in_specs=[
            perb3(SEQ, DIM),
            full2(SEQ, DIM),
            full2(2, DIM),
            perb3(SEQ, 1),
            perl3(DIM, DIM),
            perl3(DIM, DIM),
            perl3(DIM, DIM),
            perl3(DIM, DIM),
            perl3(DIM, FF),
            perl3(1, FF),
            perl3(FF, DIM),
            perl3(1, DIM),
            perl3(1, DIM),
            perl3(1, DIM),
            perl3(1, DIM),
            perl3(1, DIM),
            full2(1, DIM),
            full2(1, DIM),
            full2(DIM, 128),
            full2(1, 128),
        ],
        out_specs=pl.BlockSpec((1, SEQ, 2), lambda l, b: (b, 0, 0)),
        out_shape=jax.ShapeDtypeStruct((B, SEQ, 2), jnp.float32),
        scratch_shapes=[pltpu.VMEM((B, SEQ, DIM), jnp.float32)],
        compiler_params=pltpu.CompilerParams(
            dimension_semantics=("arbitrary", "arbitrary")),
    )(rows, pos_emb, seg_emb, sidf,
      Wq.astype(jnp.bfloat16), Wk.astype(jnp.bfloat16),
      Wv.astype(jnp.bfloat16), Wo.astype(jnp.bfloat16),
      W1.astype(jnp.bfloat16), b1.reshape(DEPTH, 1, FF),
      W2.astype(jnp.bfloat16), b2.reshape(DEPTH, 1, DIM),
      ln1_g.reshape(DEPTH, 1, DIM), ln1_b.reshape(DEPTH, 1, DIM),
      ln2_g.reshape(DEPTH, 1, DIM), ln2_b.reshape(DEPTH, 1, DIM),
      lnf_g.reshape(1, DIM), lnf_b.reshape(1, DIM), qa_Wp, qa_bp)

    return logits[:, :, 0], logits[:, :, 1]
